# Initial kernel scaffold; baseline (speedup 1.0000x reference)
import jax
import jax.numpy as jnp
from jax.experimental import pallas as pl


def kernel(x, edge_index, W0, a_src0, a_dst0, b0, W1, a_src1, a_dst1, b1):
    raise NotImplementedError("write your pallas kernel here")


# speedup vs baseline: 7.3702x; 7.3702x over previous
import jax
import jax.numpy as jnp
from jax import lax
from jax.experimental import pallas as pl
from jax.experimental.pallas import tpu as pltpu
from jax.experimental.pallas import tpu_sc as plsc

N = 10000
D = 256
DQ = 64
E = 160000
EP = 163840
NT = 16
NCH = 80
C = 128
EPT = EP // NT
RPT = 624
NREM = N - NT * RPT

_f32 = jnp.float32
_i32 = jnp.int32


BN = 2000
NSTEP = N // BN


def _store_quarters(hq_ref, h):
    for q in range(4):
        hq_ref[q] = h[:, q * DQ:(q + 1) * DQ]


def _logits(h, asrc_ref, adst_ref, as_ref, ad_ref, m_ref, sms):
    a_s = jnp.dot(h, asrc_ref[...], preferred_element_type=_f32)
    a_d = jnp.dot(h, adst_ref[...], preferred_element_type=_f32)
    as_ref[...] = a_s
    ad_ref[...] = a_d
    i = pl.program_id(0)
    bs = jnp.max(a_s)
    bd = jnp.max(a_d)

    @pl.when(i == 0)
    def _():
        sms[0] = bs
        sms[1] = bd

    @pl.when(i > 0)
    def _():
        sms[0] = jnp.maximum(sms[0], bs)
        sms[1] = jnp.maximum(sms[1], bd)

    @pl.when(i == NSTEP - 1)
    def _():
        m_ref[...] = jnp.full((8, 128),
                              jnp.maximum(sms[0] + sms[1], 0.0), _f32)


def _dense_first(x_ref, w_ref, asrc_ref, adst_ref,
                 hq_ref, as_ref, ad_ref, m_ref, sms):
    h = jnp.dot(x_ref[...], w_ref[...], preferred_element_type=_f32)
    _store_quarters(hq_ref, h)
    _logits(h, asrc_ref, adst_ref, as_ref, ad_ref, m_ref, sms)


def _normed_input(u_ref, den_ref, b_ref):
    inv = 1.0 / (den_ref[...][:, 0:1] + 1e-16)
    xs = []
    for q in range(4):
        v = u_ref[q] * inv + b_ref[...][:, q * DQ:(q + 1) * DQ]
        xs.append(jnp.where(v > 0, v, jnp.exp(v) - 1.0))
    return xs


def _dense_mid(u_ref, den_ref, b_ref, w_ref, asrc_ref, adst_ref,
               hq_ref, as_ref, ad_ref, m_ref, sms):
    xs = _normed_input(u_ref, den_ref, b_ref)
    h = sum(jnp.dot(xq, w_ref[q * DQ:(q + 1) * DQ, :],
                    preferred_element_type=_f32)
            for q, xq in enumerate(xs))
    _store_quarters(hq_ref, h)
    _logits(h, asrc_ref, adst_ref, as_ref, ad_ref, m_ref, sms)


def _dense_last(u_ref, den_ref, b_ref, out_ref):
    xs = _normed_input(u_ref, den_ref, b_ref)
    for q in range(4):
        out_ref[:, q * DQ:(q + 1) * DQ] = xs[q]


_SPEC_U = pl.BlockSpec((4, BN, DQ), lambda i: (0, i, 0))
_SPEC_DEN = pl.BlockSpec((BN, 16), lambda i: (i, 0))
_SPEC_B = pl.BlockSpec((1, D), lambda i: (0, 0))
_SPEC_W = pl.BlockSpec((D, D), lambda i: (0, 0))
_SPEC_AV = pl.BlockSpec((D, 1), lambda i: (0, 0))
_SPEC_OUT = (pl.BlockSpec((4, BN, DQ), lambda i: (0, i, 0)),
             pl.BlockSpec((BN, 1), lambda i: (i, 0)),
             pl.BlockSpec((BN, 1), lambda i: (i, 0)),
             pl.BlockSpec((8, 128), lambda i: (0, 0)))


def _sc_edge_body(hq_ref, as_ref, ad_ref, src_ref, dst_ref, m_ref,
                  u_ref, den_ref,
                  asv, adv, idxv, dstv, exv, rows, rb16, zb, zd, mv,
                  u_sh, den_sh):
    c = lax.axis_index("c")
    s = lax.axis_index("s")
    zeros16 = jnp.zeros((16,), _f32)
    z16i = jnp.zeros((16,), _i32)
    iota16 = lax.iota(_i32, 16)

    def _zrow(r, _):
        for q in range(DQ // 16):
            zb[r, pl.ds(q * 16, 16)] = zeros16
        rb16[r, pl.ds(0, 16)] = zeros16
        return 0
    lax.fori_loop(0, 128, _zrow, 0)

    def _zdrow(r, _):
        zd[r, pl.ds(0, 16)] = zeros16
        return 0
    lax.fori_loop(0, RPT, _zdrow, 0)

    base_r = s * RPT

    @pl.when(c == 0)
    def _():
        pltpu.sync_copy(zd, den_sh.at[pl.ds(base_r, RPT)])

        @pl.when(s == NT - 1)
        def _():
            pltpu.sync_copy(zd.at[pl.ds(0, NREM)],
                            den_sh.at[pl.ds(NT * RPT, NREM)])

    pltpu.sync_copy(as_ref, asv)
    pltpu.sync_copy(ad_ref, adv)
    pltpu.sync_copy(m_ref, mv)
    pltpu.sync_copy(src_ref.at[s], idxv)
    pltpu.sync_copy(dst_ref.at[s], dstv)

    plsc.subcore_barrier()

    mval = mv[0, pl.ds(0, 16)]
    ebase = s * EPT

    def _s1(k, _):
        for j in range(8):
            sv = idxv[k, pl.ds(j * 16, 16)]
            dv = dstv[k, pl.ds(j * 16, 16)]
            a1 = plsc.load_gather(asv, [sv])
            a2 = plsc.load_gather(adv, [dv])
            e = a1 + a2
            e = jnp.where(e >= 0.0, e, 0.2 * e)
            ex = jnp.exp(e - mval)
            g = ebase + k * C + j * 16 + iota16
            ex = jnp.where(g < E, ex, 0.0)
            exv[k, pl.ds(j * 16, 16)] = ex
            plsc.store_scatter(rb16, [j * 16 + iota16, z16i], ex)

        @pl.when(c == 0)
        def _():
            pltpu.sync_copy(rb16, den_sh.at[dstv.at[k]], add=True)
        return 0
    lax.fori_loop(0, NCH, _s1, 0)

    def _scale(k):
        def _srow(r, _):
            exb = plsc.load_gather(exv, [z16i + k, z16i + r])
            for q in range(DQ // 16):
                rows[r, pl.ds(q * 16, 16)] = rows[r, pl.ds(q * 16, 16)] * exb
            return 0
        lax.fori_loop(0, C, _srow, 0)

    for p in range(2):
        q = c * 2 + p

        for i in range(4):
            pltpu.sync_copy(zb, u_sh.at[pl.ds(base_r + i * 128, 128)])
        pltpu.sync_copy(zb.at[pl.ds(0, RPT - 512)],
                        u_sh.at[pl.ds(base_r + 512, RPT - 512)])

        @pl.when(s == NT - 1)
        def _():
            pltpu.sync_copy(zb.at[pl.ds(0, NREM)],
                            u_sh.at[pl.ds(NT * RPT, NREM)])

        plsc.subcore_barrier()

        def _s2(k, _):
            pltpu.sync_copy(hq_ref.at[q].at[idxv.at[k]], rows)
            _scale(k)
            pltpu.sync_copy(rows, u_sh.at[dstv.at[k]], add=True)
            return 0
        lax.fori_loop(0, NCH, _s2, 0)

        plsc.subcore_barrier()

        pltpu.sync_copy(u_sh.at[pl.ds(base_r, RPT)],
                        u_ref.at[q, pl.ds(base_r, RPT)])

        @pl.when(s == NT - 1)
        def _():
            pltpu.sync_copy(u_sh.at[pl.ds(NT * RPT, NREM)],
                            u_ref.at[q, pl.ds(NT * RPT, NREM)])

    @pl.when(c == 0)
    def _():
        pltpu.sync_copy(den_sh.at[pl.ds(base_r, RPT)],
                        den_ref.at[pl.ds(base_r, RPT)])

        @pl.when(s == NT - 1)
        def _():
            pltpu.sync_copy(den_sh.at[pl.ds(NT * RPT, NREM)],
                            den_ref.at[pl.ds(NT * RPT, NREM)])


def _sc_edge(hq, a_s, a_d, src3, dst3, m):
    mesh = plsc.VectorSubcoreMesh(core_axis_name="c", subcore_axis_name="s")
    f = pl.kernel(
        _sc_edge_body,
        out_type=(jax.ShapeDtypeStruct((4, N, DQ), _f32),
                  jax.ShapeDtypeStruct((N, 16), _f32)),
        mesh=mesh,
        compiler_params=pltpu.CompilerParams(needs_layout_passes=False,
                                             use_tc_tiling_on_sc=False),
        scratch_types=[
            pltpu.VMEM((N,), _f32),
            pltpu.VMEM((N,), _f32),
            pltpu.VMEM((NCH, C), _i32),
            pltpu.VMEM((NCH, C), _i32),
            pltpu.VMEM((NCH, C), _f32),
            pltpu.VMEM((C, DQ), _f32),
            pltpu.VMEM((C, 16), _f32),
            pltpu.VMEM((128, DQ), _f32),
            pltpu.VMEM((RPT, 16), _f32),
            pltpu.VMEM((8, 128), _f32),
            pltpu.VMEM_SHARED((N, DQ), _f32),
            pltpu.VMEM_SHARED((N, 16), _f32),
        ],
    )
    return f(hq, a_s.reshape(N), a_d.reshape(N), src3, dst3, m)


def kernel(x, edge_index, W0, a_src0, a_dst0, b0, W1, a_src1, a_dst1, b1):
    ei = edge_index.astype(_i32)
    src3 = jnp.pad(ei[0], (0, EP - E)).reshape(NT, NCH, C)
    dst3 = jnp.pad(ei[1], (0, EP - E)).reshape(NT, NCH, C)

    h_sd = jax.ShapeDtypeStruct
    dense_out = (h_sd((4, N, DQ), _f32),
                 h_sd((N, 1), _f32), h_sd((N, 1), _f32),
                 h_sd((8, 128), _f32))

    hq, a_s, a_d, m = pl.pallas_call(
        _dense_first, out_shape=dense_out,
        grid=(NSTEP,),
        in_specs=[pl.BlockSpec((BN, D), lambda i: (i, 0)),
                  _SPEC_W, _SPEC_AV, _SPEC_AV],
        out_specs=_SPEC_OUT,
        scratch_shapes=[pltpu.SMEM((2,), _f32)],
    )(x, W0, a_src0.reshape(D, 1), a_dst0.reshape(D, 1))

    u, den = _sc_edge(hq, a_s, a_d, src3, dst3, m)

    hq, a_s, a_d, m = pl.pallas_call(
        _dense_mid, out_shape=dense_out,
        grid=(NSTEP,),
        in_specs=[_SPEC_U, _SPEC_DEN, _SPEC_B, _SPEC_W, _SPEC_AV, _SPEC_AV],
        out_specs=_SPEC_OUT,
        scratch_shapes=[pltpu.SMEM((2,), _f32)],
    )(u, den, b0.reshape(1, D), W1, a_src1.reshape(D, 1), a_dst1.reshape(D, 1))

    u, den = _sc_edge(hq, a_s, a_d, src3, dst3, m)

    out = pl.pallas_call(
        _dense_last, out_shape=h_sd((N, D), _f32),
        grid=(NSTEP,),
        in_specs=[_SPEC_U, _SPEC_DEN, _SPEC_B],
        out_specs=pl.BlockSpec((BN, D), lambda i: (i, 0)),
    )(u, den, b1.reshape(1, D))
    return out


# speedup vs baseline: 7.9335x; 1.0764x over previous
import jax
import jax.numpy as jnp
from jax import lax
from jax.experimental import pallas as pl
from jax.experimental.pallas import tpu as pltpu
from jax.experimental.pallas import tpu_sc as plsc

N = 10000
D = 256
DQ = 32
NQ = 8
NP = 4
E = 160000
EP = 163840
NT = 16
NCH = 80
C = 128
EPT = EP // NT
RPT = 624
NREM = N - NT * RPT

_f32 = jnp.float32
_i32 = jnp.int32


BN = 2000
NSTEP = N // BN


def _store_quarters(hq_ref, h):
    for q in range(NQ):
        hq_ref[q] = h[:, q * DQ:(q + 1) * DQ]


def _logits(h, asrc_ref, adst_ref, as_ref, ad_ref, m_ref, sms):
    a_s = jnp.dot(h, asrc_ref[...], preferred_element_type=_f32)
    a_d = jnp.dot(h, adst_ref[...], preferred_element_type=_f32)
    as_ref[...] = a_s
    ad_ref[...] = a_d
    i = pl.program_id(0)
    bs = jnp.max(a_s)
    bd = jnp.max(a_d)

    @pl.when(i == 0)
    def _():
        sms[0] = bs
        sms[1] = bd

    @pl.when(i > 0)
    def _():
        sms[0] = jnp.maximum(sms[0], bs)
        sms[1] = jnp.maximum(sms[1], bd)

    @pl.when(i == NSTEP - 1)
    def _():
        m_ref[...] = jnp.full((8, 128),
                              jnp.maximum(sms[0] + sms[1], 0.0), _f32)


def _dense_first(x_ref, w_ref, asrc_ref, adst_ref,
                 hq_ref, as_ref, ad_ref, m_ref, sms):
    h = jnp.dot(x_ref[...], w_ref[...], preferred_element_type=_f32)
    _store_quarters(hq_ref, h)
    _logits(h, asrc_ref, adst_ref, as_ref, ad_ref, m_ref, sms)


def _normed_input(u_ref, den_ref, b_ref):
    inv = 1.0 / (den_ref[...][:, 0:1] + 1e-16)
    xs = []
    for q in range(NQ):
        v = u_ref[q] * inv + b_ref[...][:, q * DQ:(q + 1) * DQ]
        xs.append(jnp.where(v > 0, v, jnp.exp(v) - 1.0))
    return xs


def _dense_mid(u_ref, den_ref, b_ref, w_ref, asrc_ref, adst_ref,
               hq_ref, as_ref, ad_ref, m_ref, sms):
    xs = _normed_input(u_ref, den_ref, b_ref)
    h = sum(jnp.dot(xq, w_ref[q * DQ:(q + 1) * DQ, :],
                    preferred_element_type=_f32)
            for q, xq in enumerate(xs))
    _store_quarters(hq_ref, h)
    _logits(h, asrc_ref, adst_ref, as_ref, ad_ref, m_ref, sms)


def _dense_last(u_ref, den_ref, b_ref, out_ref):
    xs = _normed_input(u_ref, den_ref, b_ref)
    for q in range(NQ):
        out_ref[:, q * DQ:(q + 1) * DQ] = xs[q]


_SPEC_U = pl.BlockSpec((NQ, BN, DQ), lambda i: (0, i, 0))
_SPEC_DEN = pl.BlockSpec((BN, 16), lambda i: (i, 0))
_SPEC_B = pl.BlockSpec((1, D), lambda i: (0, 0))
_SPEC_W = pl.BlockSpec((D, D), lambda i: (0, 0))
_SPEC_AV = pl.BlockSpec((D, 1), lambda i: (0, 0))
_SPEC_OUT = (pl.BlockSpec((NQ, BN, DQ), lambda i: (0, i, 0)),
             pl.BlockSpec((BN, 1), lambda i: (i, 0)),
             pl.BlockSpec((BN, 1), lambda i: (i, 0)),
             pl.BlockSpec((8, 128), lambda i: (0, 0)))


def _sc_edge_body(hq_ref, as_ref, ad_ref, src_ref, dst_ref, m_ref,
                  u_ref, den_ref,
                  asv, adv, idxv, dstv, exv, rows, rows2, gsem0, gsem1,
                  rb16, zb, zd, mv, u_sh, den_sh):
    c = lax.axis_index("c")
    s = lax.axis_index("s")
    zeros16 = jnp.zeros((16,), _f32)
    z16i = jnp.zeros((16,), _i32)
    iota16 = lax.iota(_i32, 16)

    def _zrow(r, _):
        for q in range(DQ // 16):
            zb[r, pl.ds(q * 16, 16)] = zeros16
        rb16[r, pl.ds(0, 16)] = zeros16
        return 0
    lax.fori_loop(0, 128, _zrow, 0)

    def _zdrow(r, _):
        zd[r, pl.ds(0, 16)] = zeros16
        return 0
    lax.fori_loop(0, RPT, _zdrow, 0)

    base_r = s * RPT

    @pl.when(c == 0)
    def _():
        pltpu.sync_copy(zd, den_sh.at[pl.ds(base_r, RPT)])

        @pl.when(s == NT - 1)
        def _():
            pltpu.sync_copy(zd.at[pl.ds(0, NREM)],
                            den_sh.at[pl.ds(NT * RPT, NREM)])

    pltpu.sync_copy(as_ref, asv)
    pltpu.sync_copy(ad_ref, adv)
    pltpu.sync_copy(m_ref, mv)
    pltpu.sync_copy(src_ref.at[s], idxv)
    pltpu.sync_copy(dst_ref.at[s], dstv)

    plsc.subcore_barrier()

    mval = mv[0, pl.ds(0, 16)]
    ebase = s * EPT

    def _s1(k, _):
        for j in range(8):
            sv = idxv[k, pl.ds(j * 16, 16)]
            dv = dstv[k, pl.ds(j * 16, 16)]
            a1 = plsc.load_gather(asv, [sv])
            a2 = plsc.load_gather(adv, [dv])
            e = a1 + a2
            e = jnp.where(e >= 0.0, e, 0.2 * e)
            ex = jnp.exp(e - mval)
            g = ebase + k * C + j * 16 + iota16
            ex = jnp.where(g < E, ex, 0.0)
            exv[k, pl.ds(j * 16, 16)] = ex
            plsc.store_scatter(rb16, [j * 16 + iota16, z16i], ex)

        @pl.when(c == 0)
        def _():
            pltpu.sync_copy(rb16, den_sh.at[dstv.at[k]], add=True)
        return 0
    lax.fori_loop(0, NCH, _s1, 0)

    def _scale(buf, k):
        def _srow(r, _):
            exb = plsc.load_gather(exv, [z16i + k, z16i + r])
            for q in range(DQ // 16):
                buf[r, pl.ds(q * 16, 16)] = buf[r, pl.ds(q * 16, 16)] * exb
            return 0
        lax.fori_loop(0, C, _srow, 0)

    for p in range(NP):
        q = c * NP + p

        for i in range(4):
            pltpu.sync_copy(zb, u_sh.at[pl.ds(base_r + i * 128, 128)])
        pltpu.sync_copy(zb.at[pl.ds(0, RPT - 512)],
                        u_sh.at[pl.ds(base_r + 512, RPT - 512)])

        @pl.when(s == NT - 1)
        def _():
            pltpu.sync_copy(zb.at[pl.ds(0, NREM)],
                            u_sh.at[pl.ds(NT * RPT, NREM)])

        plsc.subcore_barrier()

        def _gather(k, buf, sem):
            return pltpu.async_copy(hq_ref.at[q].at[idxv.at[k]], buf, sem)

        _gather(0, rows, gsem0).wait()

        def _s2(t, _):
            k0 = 2 * t
            d1 = _gather(k0 + 1, rows2, gsem1)
            _scale(rows, k0)
            pltpu.sync_copy(rows, u_sh.at[dstv.at[k0]], add=True)
            d1.wait()

            @pl.when(t < NCH // 2 - 1)
            def _():
                _gather(k0 + 2, rows, gsem0)
            _scale(rows2, k0 + 1)
            pltpu.sync_copy(rows2, u_sh.at[dstv.at[k0 + 1]], add=True)

            @pl.when(t < NCH // 2 - 1)
            def _():
                pltpu.make_async_copy(hq_ref.at[q].at[idxv.at[k0 + 2]],
                                      rows, gsem0).wait()
            return 0
        lax.fori_loop(0, NCH // 2, _s2, 0)

        plsc.subcore_barrier()

        pltpu.sync_copy(u_sh.at[pl.ds(base_r, RPT)],
                        u_ref.at[q, pl.ds(base_r, RPT)])

        @pl.when(s == NT - 1)
        def _():
            pltpu.sync_copy(u_sh.at[pl.ds(NT * RPT, NREM)],
                            u_ref.at[q, pl.ds(NT * RPT, NREM)])

    @pl.when(c == 0)
    def _():
        pltpu.sync_copy(den_sh.at[pl.ds(base_r, RPT)],
                        den_ref.at[pl.ds(base_r, RPT)])

        @pl.when(s == NT - 1)
        def _():
            pltpu.sync_copy(den_sh.at[pl.ds(NT * RPT, NREM)],
                            den_ref.at[pl.ds(NT * RPT, NREM)])


def _sc_edge(hq, a_s, a_d, src3, dst3, m):
    mesh = plsc.VectorSubcoreMesh(core_axis_name="c", subcore_axis_name="s")
    f = pl.kernel(
        _sc_edge_body,
        out_type=(jax.ShapeDtypeStruct((NQ, N, DQ), _f32),
                  jax.ShapeDtypeStruct((N, 16), _f32)),
        mesh=mesh,
        compiler_params=pltpu.CompilerParams(needs_layout_passes=False,
                                             use_tc_tiling_on_sc=False),
        scratch_types=[
            pltpu.VMEM((N,), _f32),
            pltpu.VMEM((N,), _f32),
            pltpu.VMEM((NCH, C), _i32),
            pltpu.VMEM((NCH, C), _i32),
            pltpu.VMEM((NCH, C), _f32),
            pltpu.VMEM((C, DQ), _f32),
            pltpu.VMEM((C, DQ), _f32),
            pltpu.SemaphoreType.DMA,
            pltpu.SemaphoreType.DMA,
            pltpu.VMEM((C, 16), _f32),
            pltpu.VMEM((128, DQ), _f32),
            pltpu.VMEM((RPT, 16), _f32),
            pltpu.VMEM((8, 128), _f32),
            pltpu.VMEM_SHARED((N, DQ), _f32),
            pltpu.VMEM_SHARED((N, 16), _f32),
        ],
    )
    return f(hq, a_s.reshape(N), a_d.reshape(N), src3, dst3, m)


def kernel(x, edge_index, W0, a_src0, a_dst0, b0, W1, a_src1, a_dst1, b1):
    ei = edge_index.astype(_i32)
    src3 = jnp.pad(ei[0], (0, EP - E)).reshape(NT, NCH, C)
    dst3 = jnp.pad(ei[1], (0, EP - E)).reshape(NT, NCH, C)

    h_sd = jax.ShapeDtypeStruct
    dense_out = (h_sd((NQ, N, DQ), _f32),
                 h_sd((N, 1), _f32), h_sd((N, 1), _f32),
                 h_sd((8, 128), _f32))

    hq, a_s, a_d, m = pl.pallas_call(
        _dense_first, out_shape=dense_out,
        grid=(NSTEP,),
        in_specs=[pl.BlockSpec((BN, D), lambda i: (i, 0)),
                  _SPEC_W, _SPEC_AV, _SPEC_AV],
        out_specs=_SPEC_OUT,
        scratch_shapes=[pltpu.SMEM((2,), _f32)],
    )(x, W0, a_src0.reshape(D, 1), a_dst0.reshape(D, 1))

    u, den = _sc_edge(hq, a_s, a_d, src3, dst3, m)

    hq, a_s, a_d, m = pl.pallas_call(
        _dense_mid, out_shape=dense_out,
        grid=(NSTEP,),
        in_specs=[_SPEC_U, _SPEC_DEN, _SPEC_B, _SPEC_W, _SPEC_AV, _SPEC_AV],
        out_specs=_SPEC_OUT,
        scratch_shapes=[pltpu.SMEM((2,), _f32)],
    )(u, den, b0.reshape(1, D), W1, a_src1.reshape(D, 1), a_dst1.reshape(D, 1))

    u, den = _sc_edge(hq, a_s, a_d, src3, dst3, m)

    out = pl.pallas_call(
        _dense_last, out_shape=h_sd((N, D), _f32),
        grid=(NSTEP,),
        in_specs=[_SPEC_U, _SPEC_DEN, _SPEC_B],
        out_specs=pl.BlockSpec((BN, D), lambda i: (i, 0)),
    )(u, den, b1.reshape(1, D))
    return out
